# baseline (speedup 1.0000x reference)
import functools

import jax
import jax.numpy as jnp
from jax import lax
from jax.experimental import pallas as pl
from jax.experimental.pallas import tpu as pltpu
from jax.experimental.pallas import tpu_sc as plsc

_NC, _NS, _L = 2, 16, 16
_NW = _NC * _NS
_R = 1000
_CB = 128


def _hist_call(o_idx, n_obj):
    t = o_idx.shape[0]
    nch = t // _CB
    full, extra = nch // _NW, nch % _NW
    mesh = plsc.VectorSubcoreMesh(core_axis_name="c", subcore_axis_name="s")

    @functools.partial(
        pl.kernel, mesh=mesh,
        out_type=jax.ShapeDtypeStruct((_NW, n_obj), jnp.float32),
        compiler_params=pltpu.CompilerParams(needs_layout_passes=False),
        scratch_types=[
            pltpu.VMEM((n_obj,), jnp.float32),
            pltpu.VMEM((_CB,), jnp.int32),
        ],
    )
    def hist_k(o_hbm, out_hbm, cnt_v, idx_v):
        wid = lax.axis_index("s") * _NC + lax.axis_index("c")
        zero16 = jnp.zeros((_L,), jnp.float32)

        def zb(i, c):
            cnt_v[pl.ds(i * _L, _L)] = zero16
            return c
        lax.fori_loop(0, n_obj // _L, zb, 0)

        one16 = jnp.ones((_L,), jnp.float32)
        nmine = full + jnp.where(wid < extra, 1, 0)

        def cb(j, c):
            eb = (wid + j * _NW) * _CB
            pltpu.sync_copy(o_hbm.at[pl.ds(eb, _CB)], idx_v)
            for i in range(_CB // _L):
                plsc.addupdate_scatter(cnt_v, [idx_v[pl.ds(i * _L, _L)]], one16)
            return c
        lax.fori_loop(0, nmine, cb, 0)
        pltpu.sync_copy(cnt_v, out_hbm.at[wid])

    return hist_k(o_idx)


def _gs_call(g_lin, g_gat, s_idx, o_idx, b_vec, pred_ref,
             n_obj, ebase, lin_base, half_t, head):
    d = g_lin.shape[1]
    t = half_t
    cz = 64
    nch = t // cz
    full, extra = nch // _NW, nch % _NW
    rz = 624
    rtail = n_obj - _NS * rz
    mesh = plsc.VectorSubcoreMesh(core_axis_name="c", subcore_axis_name="s")
    if head:
        out_type = (jax.ShapeDtypeStruct((n_obj + _R, d), jnp.float32),
                    jax.ShapeDtypeStruct((_NC, n_obj, d), jnp.float32))
    else:
        out_type = jax.ShapeDtypeStruct((_NC, n_obj, d), jnp.float32)

    @functools.partial(
        pl.kernel, mesh=mesh,
        out_type=out_type,
        compiler_params=pltpu.CompilerParams(needs_layout_passes=False),
        scratch_types=[
            pltpu.VMEM((cz,), jnp.int32),
            pltpu.VMEM((cz,), jnp.int32),
            pltpu.VMEM((cz, d), jnp.float32),
            pltpu.VMEM((cz, d), jnp.float32),
            pltpu.VMEM((cz,), jnp.int32),
            pltpu.VMEM((cz,), jnp.int32),
            pltpu.VMEM((cz, d), jnp.float32),
            pltpu.VMEM((cz, d), jnp.float32),
            pltpu.VMEM((d,), jnp.float32),
            pltpu.VMEM_SHARED((n_obj, d), jnp.float32),
            pltpu.SemaphoreType.DMA,
            pltpu.SemaphoreType.DMA,
            pltpu.SemaphoreType.DMA,
            pltpu.SemaphoreType.DMA,
            pltpu.SemaphoreType.DMA,
            pltpu.SemaphoreType.DMA,
        ],
    )
    def gs_k(glin_hbm, ggat_hbm, s_hbm, o_hbm, b_hbm, pred_hbm, *rest):
        if head:
            (gh_hbm, acc_hbm,
             sidx_a, oidx_a, grows_a, rows_a,
             sidx_b, oidx_b, grows_b, rows_b,
             b_v, acc_sh, ld_a, ld_b, gat, st_a, st_b, sc) = rest
        else:
            (acc_hbm,
             sidx_a, oidx_a, grows_a, rows_a,
             sidx_b, oidx_b, grows_b, rows_b,
             b_v, acc_sh, ld_a, ld_b, gat, st_a, st_b, sc) = rest
            gh_hbm = None
        cid = lax.axis_index("c")
        sid = lax.axis_index("s")
        wid = sid * _NC + cid

        pltpu.sync_copy(b_hbm, b_v)
        b8 = [b_v[pl.ds(i * _L, _L)] for i in range(d // _L)]

        zf = jnp.zeros((_L,), jnp.float32)

        def zb(r, c):
            for i in range(d // _L):
                grows_a[r, pl.ds(i * _L, _L)] = zf
            return c
        lax.fori_loop(0, cz, zb, 0)
        nzc, ztail = rz // cz, rz % cz
        zhs = [pltpu.async_copy(grows_a,
                                acc_sh.at[pl.ds(sid * rz + z * cz, cz)], ld_b)
               for z in range(nzc)]
        if ztail:
            zhs.append(pltpu.async_copy(
                grows_a.at[pl.ds(0, ztail)],
                acc_sh.at[pl.ds(sid * rz + nzc * cz, ztail)], ld_b))
        if rtail:
            @pl.when(sid == 0)
            def _():
                pltpu.async_copy(grows_a.at[pl.ds(0, rtail)],
                                 acc_sh.at[pl.ds(_NS * rz, rtail)],
                                 ld_b).wait()
        for h in zhs:
            h.wait()
        plsc.subcore_barrier()

        nmine = full + jnp.where(wid < extra, 1, 0)

        def loads(j, sidx_v, oidx_v, grows_v, ld):
            eb = (wid + j * _NW) * cz
            pltpu.async_copy(s_hbm.at[pl.ds(ebase + eb, cz)], sidx_v, ld)
            pltpu.async_copy(o_hbm.at[pl.ds(ebase + eb, cz)], oidx_v, ld)
            pltpu.async_copy(glin_hbm.at[pl.ds(ebase - lin_base + eb, cz)],
                             grows_v, ld)

        def wait_loads(j, sidx_v, oidx_v, grows_v, ld):
            eb = (wid + j * _NW) * cz
            pltpu.make_async_copy(
                s_hbm.at[pl.ds(ebase + eb, cz)], sidx_v, ld).wait()
            pltpu.make_async_copy(
                o_hbm.at[pl.ds(ebase + eb, cz)], oidx_v, ld).wait()
            pltpu.make_async_copy(
                glin_hbm.at[pl.ds(ebase - lin_base + eb, cz)],
                grows_v, ld).wait()

        def fuse(rows_v, grows_v):
            def fb(r, c):
                for i in range(d // _L):
                    sl = pl.ds(i * _L, _L)
                    rows_v[r, sl] = ((rows_v[r, sl] + grows_v[r, sl])
                                     * 0.7071067811865476 + b8[i])
                return c
            lax.fori_loop(0, cz, fb, 0)

        def finish(j, sidx_v, oidx_v, grows_v, rows_v, st):
            eb = (wid + j * _NW) * cz
            hs = pltpu.async_copy(grows_v, acc_sh.at[oidx_v], sc, add=True)
            if head:
                @pl.when(eb < head)
                def _():
                    pltpu.async_copy(rows_v, gh_hbm.at[pl.ds(eb, cz)], st)

                @pl.when(eb >= head)
                def _():
                    fuse(rows_v, grows_v)
                    pltpu.async_copy(
                        rows_v,
                        pred_hbm.at[pl.ds(ebase + eb - n_obj, cz)], st)
            else:
                fuse(rows_v, grows_v)
                pltpu.async_copy(
                    rows_v, pred_hbm.at[pl.ds(ebase + eb - n_obj, cz)], st)
            hs.wait()

        def wait_store(j, rows_v, st):
            eb = (wid + j * _NW) * cz
            if head:
                @pl.when(eb < head)
                def _():
                    pltpu.make_async_copy(
                        rows_v, gh_hbm.at[pl.ds(eb, cz)], st).wait()

                @pl.when(eb >= head)
                def _():
                    pltpu.make_async_copy(
                        rows_v,
                        pred_hbm.at[pl.ds(ebase + eb - n_obj, cz)], st).wait()
            else:
                pltpu.make_async_copy(
                    rows_v,
                    pred_hbm.at[pl.ds(ebase + eb - n_obj, cz)], st).wait()

        def stage(j, cur, oth):
            (sidx_c, oidx_c, grows_c, rows_c, ld_c, st_c) = cur
            (sidx_o, oidx_o, grows_o, rows_o, ld_o, st_o) = oth

            @pl.when(j >= 2)
            def _():
                wait_store(j - 2, rows_c, st_c)

            wait_loads(j, sidx_c, oidx_c, grows_c, ld_c)
            hg = pltpu.async_copy(ggat_hbm.at[sidx_c], rows_c, gat)

            @pl.when(j >= 1)
            def _():
                finish(j - 1, sidx_o, oidx_o, grows_o, rows_o, st_o)

            @pl.when(j + 1 < nmine)
            def _():
                loads(j + 1, sidx_o, oidx_o, grows_o, ld_o)

            hg.wait()

        buf_a = (sidx_a, oidx_a, grows_a, rows_a, ld_a, st_a)
        buf_b = (sidx_b, oidx_b, grows_b, rows_b, ld_b, st_b)

        loads(0, sidx_a, oidx_a, grows_a, ld_a)

        def cb(j, c):
            @pl.when((j & 1) == 0)
            def _():
                stage(j, buf_a, buf_b)

            @pl.when((j & 1) == 1)
            def _():
                stage(j, buf_b, buf_a)
            return c
        lax.fori_loop(0, nmine, cb, 0)

        @pl.when(((nmine - 1) & 1) == 0)
        def _():
            finish(nmine - 1, sidx_a, oidx_a, grows_a, rows_a, st_a)
            wait_store(nmine - 2, rows_b, st_b)
            wait_store(nmine - 1, rows_a, st_a)

        @pl.when(((nmine - 1) & 1) == 1)
        def _():
            finish(nmine - 1, sidx_b, oidx_b, grows_b, rows_b, st_b)
            wait_store(nmine - 2, rows_a, st_a)
            wait_store(nmine - 1, rows_b, st_b)

        plsc.subcore_barrier()
        pltpu.sync_copy(acc_sh.at[pl.ds(sid * rz, rz)],
                        acc_hbm.at[cid, pl.ds(sid * rz, rz)])
        if rtail:
            @pl.when(sid == 0)
            def _():
                pltpu.sync_copy(acc_sh.at[pl.ds(_NS * rz, rtail)],
                                acc_hbm.at[cid, pl.ds(_NS * rz, rtail)])

    return gs_k(g_lin, g_gat, s_idx, o_idx, b_vec, pred_ref)


def _prep_lo_pred(pred_vecs, w, n_lo, n_obj):
    d = pred_vecs.shape[1]
    ob = n_obj // _R

    def body(pred_ref, w_ref, g_ref):
        h = lax.dot_general(pred_ref[...], w_ref[...], (((1,), (1,)), ((), ())),
                            preferred_element_type=jnp.float32)
        g_ref[...] = h * 0.7071067811865476

    return pl.pallas_call(
        body,
        grid=(n_lo // _R - ob,),
        in_specs=[
            pl.BlockSpec((_R, d), lambda i: (i, 0)),
            pl.BlockSpec((d, d), lambda i: (0, 0)),
        ],
        out_specs=pl.BlockSpec((_R, d), lambda i: (i + ob, 0)),
        out_shape=jax.ShapeDtypeStruct((n_lo, d), jnp.float32),
    )(pred_vecs, w)


def _prep_obj(obj_vecs, w, cnt_t, prev):
    n_obj, d = obj_vecs.shape
    n_lo = prev.shape[0]

    def body(obj_ref, w_ref, cnt_ref, _prev_ref, g_ref):
        h = lax.dot_general(obj_ref[...], w_ref[...], (((1,), (1,)), ((), ())),
                            preferred_element_type=jnp.float32)
        cnt = jnp.sum(cnt_ref[...], axis=1, keepdims=True)
        g_ref[...] = h * lax.rsqrt(cnt + 2.0)

    return pl.pallas_call(
        body,
        grid=(n_obj // _R,),
        in_specs=[
            pl.BlockSpec((_R, d), lambda i: (i, 0)),
            pl.BlockSpec((d, d), lambda i: (0, 0)),
            pl.BlockSpec((_R, _NW), lambda i: (i, 0)),
            pl.BlockSpec((_R, d), lambda i: (i, 0)),
        ],
        out_specs=pl.BlockSpec((_R, d), lambda i: (i, 0)),
        out_shape=jax.ShapeDtypeStruct((n_lo, d), jnp.float32),
        input_output_aliases={3: 0},
    )(obj_vecs, w, cnt_t, prev)


def _prep_hi(pred_vecs, w, p0, n_hi, t_boundary):
    d = pred_vecs.shape[1]
    pb = p0 // _R
    tbb = t_boundary // _R

    def body(pred_ref, w_ref, g_ref):
        i = pl.program_id(0)
        h = lax.dot_general(pred_ref[...], w_ref[...], (((1,), (1,)), ((), ())),
                            preferred_element_type=jnp.float32)
        dinv = jnp.where(i + pb < tbb, jnp.float32(0.7071067811865476),
                         jnp.float32(1.0))
        g_ref[...] = h * dinv

    return pl.pallas_call(
        body,
        grid=(n_hi // _R,),
        in_specs=[
            pl.BlockSpec((_R, d), lambda i: (i + pb, 0)),
            pl.BlockSpec((d, d), lambda i: (0, 0)),
        ],
        out_specs=pl.BlockSpec((_R, d), lambda i: (i, 0)),
        out_shape=jax.ShapeDtypeStruct((n_hi, d), jnp.float32),
    )(pred_vecs, w)


def _combine_obj(g_lo, gath_head, accs, cnt_t, b2, n_obj):
    d = g_lo.shape[1]
    na = len(accs)

    def body(g_ref, gath_ref, *rest):
        acc_refs = rest[:na]
        cnt_ref, b_ref, out_ref = rest[na:]
        acc = acc_refs[0][0] + acc_refs[0][1]
        for a in acc_refs[1:]:
            acc = acc + a[0] + a[1]
        cnt = jnp.sum(cnt_ref[...], axis=1, keepdims=True)
        dinv = lax.rsqrt(cnt + 2.0)
        out_ref[...] = dinv * (g_ref[...] + gath_ref[...] + acc) + b_ref[...]

    return pl.pallas_call(
        body,
        grid=(n_obj // _R,),
        in_specs=[pl.BlockSpec((_R, d), lambda i: (i, 0)),
                  pl.BlockSpec((_R, d), lambda i: (i, 0))]
                 + [pl.BlockSpec((_NC, _R, d), lambda i: (0, i, 0))] * na
                 + [pl.BlockSpec((_R, _NW), lambda i: (i, 0)),
                    pl.BlockSpec((1, d), lambda i: (0, 0))],
        out_specs=pl.BlockSpec((_R, d), lambda i: (i, 0)),
        out_shape=jax.ShapeDtypeStruct((n_obj, d), jnp.float32),
    )(g_lo, gath_head, *accs, cnt_t, b2)


def _pred_tail(tail_g, b2, j0, nb, t):
    d = tail_g.shape[1]

    def body(g_ref, b_ref, out_ref):
        out_ref[...] = g_ref[...] + b_ref[...]

    return pl.pallas_call(
        body,
        grid=(nb,),
        in_specs=[pl.BlockSpec((_R, d), lambda i: (i, 0)),
                  pl.BlockSpec((1, d), lambda i: (0, 0))],
        out_specs=pl.BlockSpec((_R, d), lambda i: (i + j0, 0)),
        out_shape=jax.ShapeDtypeStruct((t, d), jnp.float32),
    )(tail_g, b2)


def _pred_fixup(g_lo, gath_head, b2, prev, n_obj, nraw):
    d = g_lo.shape[1]
    t = prev.shape[0]
    ob = n_obj // _R

    def body(g_ref, gh_ref, b_ref, prev_ref, out_ref):
        row = lax.broadcasted_iota(jnp.int32, (_R, 1), 0)
        fused = ((g_ref[...] + gh_ref[...]) * 0.7071067811865476
                 + b_ref[...])
        out_ref[...] = jnp.where(row < nraw, fused, prev_ref[...])

    return pl.pallas_call(
        body,
        grid=(1,),
        in_specs=[pl.BlockSpec((_R, d), lambda i: (ob, 0)),
                  pl.BlockSpec((_R, d), lambda i: (ob, 0)),
                  pl.BlockSpec((1, d), lambda i: (0, 0)),
                  pl.BlockSpec((_R, d), lambda i: (0, 0))],
        out_specs=pl.BlockSpec((_R, d), lambda i: (0, 0)),
        out_shape=jax.ShapeDtypeStruct((t, d), jnp.float32),
        input_output_aliases={3: 0},
    )(g_lo, gath_head, b2, prev)


def kernel(obj_vecs, pred_vecs, edges, W, b):
    n_obj, d = obj_vecs.shape
    t = pred_vecs.shape[0]
    sizes = [t * 3 // 20, t * 7 // 20, t * 7 // 20, t * 3 // 20]
    offs = [0]
    for sz in sizes:
        offs.append(offs[-1] + sz)
    cz = 64
    head = -(-n_obj // cz) * cz
    assert offs[-1] == t and sizes[0] > head
    assert all(sz % cz == 0 and sz % _R == 0 for sz in sizes)
    assert n_obj % _R == 0 and (t - n_obj) % _R == 0

    s_idx = edges[:, 0]
    o_idx = edges[:, 1]
    cnt32 = _hist_call(o_idx, n_obj)
    cnt_t = cnt32.T
    b2 = b.reshape(1, d)
    g0p = _prep_lo_pred(pred_vecs, W, sizes[0], n_obj)
    gps = [_prep_obj(obj_vecs, W, cnt_t, g0p)]
    for k in range(1, len(sizes)):
        gps.append(_prep_hi(pred_vecs, W, offs[k] - n_obj, sizes[k],
                            t - n_obj))
    tail_g = _prep_hi(pred_vecs, W, t - n_obj, n_obj, t - n_obj)
    tb = (t - n_obj) // _R
    ntail = n_obj // _R
    pred_init = _pred_tail(tail_g, b2, tb, ntail, t)
    pred_ref = jax.new_ref(pred_init)
    gath_head, acc0 = _gs_call(gps[0], gps[0], s_idx, o_idx, b, pred_ref,
                               n_obj, 0, 0, sizes[0], head)
    accs = [acc0]
    for k in range(1, len(sizes)):
        accs.append(_gs_call(gps[k], gps[0], s_idx, o_idx, b, pred_ref,
                             n_obj, offs[k], offs[k], sizes[k], 0))
    pred_val = pred_ref[...]
    out_pred = _pred_fixup(gps[0], gath_head, b2, pred_val, n_obj,
                           head - n_obj)
    out_obj = _combine_obj(gps[0], gath_head, accs, cnt_t, b2, n_obj)
    return out_obj, out_pred


# speedup vs baseline: 1.0224x; 1.0224x over previous
import functools

import jax
import jax.numpy as jnp
from jax import lax
from jax.experimental import pallas as pl
from jax.experimental.pallas import tpu as pltpu
from jax.experimental.pallas import tpu_sc as plsc

_NC, _NS, _L = 2, 16, 16
_NW = _NC * _NS
_R = 1000
_CB = 128


def _hist_call(o_idx, n_obj):
    t = o_idx.shape[0]
    nch = t // _CB
    full, extra = nch // _NW, nch % _NW
    mesh = plsc.VectorSubcoreMesh(core_axis_name="c", subcore_axis_name="s")

    @functools.partial(
        pl.kernel, mesh=mesh,
        out_type=jax.ShapeDtypeStruct((_NW, n_obj), jnp.float32),
        compiler_params=pltpu.CompilerParams(needs_layout_passes=False),
        scratch_types=[
            pltpu.VMEM((n_obj,), jnp.float32),
            pltpu.VMEM((_CB,), jnp.int32),
        ],
    )
    def hist_k(o_hbm, out_hbm, cnt_v, idx_v):
        wid = lax.axis_index("s") * _NC + lax.axis_index("c")
        zero16 = jnp.zeros((_L,), jnp.float32)

        def zb(i, c):
            cnt_v[pl.ds(i * _L, _L)] = zero16
            return c
        lax.fori_loop(0, n_obj // _L, zb, 0)

        one16 = jnp.ones((_L,), jnp.float32)
        nmine = full + jnp.where(wid < extra, 1, 0)

        def cb(j, c):
            eb = (wid + j * _NW) * _CB
            pltpu.sync_copy(o_hbm.at[pl.ds(eb, _CB)], idx_v)
            for i in range(_CB // _L):
                plsc.addupdate_scatter(cnt_v, [idx_v[pl.ds(i * _L, _L)]], one16)
            return c
        lax.fori_loop(0, nmine, cb, 0)
        pltpu.sync_copy(cnt_v, out_hbm.at[wid])

    return hist_k(o_idx)


def _gs_call(g_lin, g_gat, s_idx, o_idx, b_vec, pred_ref,
             n_obj, ebase, lin_base, half_t, head):
    d = g_lin.shape[1]
    t = half_t
    cz = 64
    nch = t // cz
    full, extra = nch // _NW, nch % _NW
    rz = 624
    rtail = n_obj - _NS * rz
    mesh = plsc.VectorSubcoreMesh(core_axis_name="c", subcore_axis_name="s")
    if head:
        out_type = (jax.ShapeDtypeStruct((n_obj + _R, d), jnp.float32),
                    jax.ShapeDtypeStruct((_NC, n_obj, d), jnp.float32))
    else:
        out_type = jax.ShapeDtypeStruct((_NC, n_obj, d), jnp.float32)

    @functools.partial(
        pl.kernel, mesh=mesh,
        out_type=out_type,
        compiler_params=pltpu.CompilerParams(needs_layout_passes=False),
        scratch_types=[
            pltpu.VMEM((cz,), jnp.int32),
            pltpu.VMEM((cz,), jnp.int32),
            pltpu.VMEM((cz, d), jnp.float32),
            pltpu.VMEM((cz, d), jnp.float32),
            pltpu.VMEM((cz,), jnp.int32),
            pltpu.VMEM((cz,), jnp.int32),
            pltpu.VMEM((cz, d), jnp.float32),
            pltpu.VMEM((cz, d), jnp.float32),
            pltpu.VMEM((d,), jnp.float32),
            pltpu.VMEM_SHARED((n_obj, d), jnp.float32),
            pltpu.SemaphoreType.DMA,
            pltpu.SemaphoreType.DMA,
            pltpu.SemaphoreType.DMA,
            pltpu.SemaphoreType.DMA,
            pltpu.SemaphoreType.DMA,
            pltpu.SemaphoreType.DMA,
        ],
    )
    def gs_k(glin_hbm, ggat_hbm, s_hbm, o_hbm, b_hbm, pred_hbm, *rest):
        if head:
            (gh_hbm, acc_hbm,
             sidx_a, oidx_a, grows_a, rows_a,
             sidx_b, oidx_b, grows_b, rows_b,
             b_v, acc_sh, ld_a, ld_b, gat, st_a, st_b, sc) = rest
        else:
            (acc_hbm,
             sidx_a, oidx_a, grows_a, rows_a,
             sidx_b, oidx_b, grows_b, rows_b,
             b_v, acc_sh, ld_a, ld_b, gat, st_a, st_b, sc) = rest
            gh_hbm = None
        cid = lax.axis_index("c")
        sid = lax.axis_index("s")
        wid = sid * _NC + cid

        pltpu.sync_copy(b_hbm, b_v)
        b8 = [b_v[pl.ds(i * _L, _L)] for i in range(d // _L)]

        zf = jnp.zeros((_L,), jnp.float32)

        def zb(r, c):
            for i in range(d // _L):
                grows_a[r, pl.ds(i * _L, _L)] = zf
            return c
        lax.fori_loop(0, cz, zb, 0)
        nzc, ztail = rz // cz, rz % cz
        zhs = [pltpu.async_copy(grows_a,
                                acc_sh.at[pl.ds(sid * rz + z * cz, cz)], ld_b)
               for z in range(nzc)]
        if ztail:
            zhs.append(pltpu.async_copy(
                grows_a.at[pl.ds(0, ztail)],
                acc_sh.at[pl.ds(sid * rz + nzc * cz, ztail)], ld_b))
        if rtail:
            @pl.when(sid == 0)
            def _():
                pltpu.async_copy(grows_a.at[pl.ds(0, rtail)],
                                 acc_sh.at[pl.ds(_NS * rz, rtail)],
                                 ld_b).wait()
        for h in zhs:
            h.wait()
        plsc.subcore_barrier()

        nmine = full + jnp.where(wid < extra, 1, 0)

        def loads(j, sidx_v, oidx_v, grows_v, ld):
            eb = (wid + j * _NW) * cz
            pltpu.async_copy(s_hbm.at[pl.ds(ebase + eb, cz)], sidx_v, ld)
            pltpu.async_copy(o_hbm.at[pl.ds(ebase + eb, cz)], oidx_v, ld)
            pltpu.async_copy(glin_hbm.at[pl.ds(ebase - lin_base + eb, cz)],
                             grows_v, ld)

        def wait_loads(j, sidx_v, oidx_v, grows_v, ld):
            eb = (wid + j * _NW) * cz
            pltpu.make_async_copy(
                s_hbm.at[pl.ds(ebase + eb, cz)], sidx_v, ld).wait()
            pltpu.make_async_copy(
                o_hbm.at[pl.ds(ebase + eb, cz)], oidx_v, ld).wait()
            pltpu.make_async_copy(
                glin_hbm.at[pl.ds(ebase - lin_base + eb, cz)],
                grows_v, ld).wait()

        def fuse(rows_v, grows_v):
            def fb(r, c):
                for i in range(d // _L):
                    sl = pl.ds(i * _L, _L)
                    rows_v[r, sl] = ((rows_v[r, sl] + grows_v[r, sl])
                                     * 0.7071067811865476 + b8[i])
                return c
            lax.fori_loop(0, cz, fb, 0)

        def finish(j, sidx_v, oidx_v, grows_v, rows_v, st):
            eb = (wid + j * _NW) * cz
            hs = pltpu.async_copy(grows_v, acc_sh.at[oidx_v], sc, add=True)
            if head:
                @pl.when(eb < head)
                def _():
                    pltpu.async_copy(rows_v, gh_hbm.at[pl.ds(eb, cz)], st)

                @pl.when(eb >= head)
                def _():
                    fuse(rows_v, grows_v)
                    pltpu.async_copy(
                        rows_v,
                        pred_hbm.at[pl.ds(ebase + eb - n_obj, cz)], st)
            else:
                fuse(rows_v, grows_v)
                pltpu.async_copy(
                    rows_v, pred_hbm.at[pl.ds(ebase + eb - n_obj, cz)], st)
            hs.wait()

        def wait_store(j, rows_v, st):
            eb = (wid + j * _NW) * cz
            if head:
                @pl.when(eb < head)
                def _():
                    pltpu.make_async_copy(
                        rows_v, gh_hbm.at[pl.ds(eb, cz)], st).wait()

                @pl.when(eb >= head)
                def _():
                    pltpu.make_async_copy(
                        rows_v,
                        pred_hbm.at[pl.ds(ebase + eb - n_obj, cz)], st).wait()
            else:
                pltpu.make_async_copy(
                    rows_v,
                    pred_hbm.at[pl.ds(ebase + eb - n_obj, cz)], st).wait()

        def stage(j, cur, oth):
            (sidx_c, oidx_c, grows_c, rows_c, ld_c, st_c) = cur
            (sidx_o, oidx_o, grows_o, rows_o, ld_o, st_o) = oth

            @pl.when(j >= 2)
            def _():
                wait_store(j - 2, rows_c, st_c)

            wait_loads(j, sidx_c, oidx_c, grows_c, ld_c)
            hg = pltpu.async_copy(ggat_hbm.at[sidx_c], rows_c, gat)

            @pl.when(j >= 1)
            def _():
                finish(j - 1, sidx_o, oidx_o, grows_o, rows_o, st_o)

            @pl.when(j + 1 < nmine)
            def _():
                loads(j + 1, sidx_o, oidx_o, grows_o, ld_o)

            hg.wait()

        buf_a = (sidx_a, oidx_a, grows_a, rows_a, ld_a, st_a)
        buf_b = (sidx_b, oidx_b, grows_b, rows_b, ld_b, st_b)

        loads(0, sidx_a, oidx_a, grows_a, ld_a)

        def cb(j, c):
            @pl.when((j & 1) == 0)
            def _():
                stage(j, buf_a, buf_b)

            @pl.when((j & 1) == 1)
            def _():
                stage(j, buf_b, buf_a)
            return c
        lax.fori_loop(0, nmine, cb, 0)

        @pl.when(((nmine - 1) & 1) == 0)
        def _():
            finish(nmine - 1, sidx_a, oidx_a, grows_a, rows_a, st_a)
            wait_store(nmine - 2, rows_b, st_b)
            wait_store(nmine - 1, rows_a, st_a)

        @pl.when(((nmine - 1) & 1) == 1)
        def _():
            finish(nmine - 1, sidx_b, oidx_b, grows_b, rows_b, st_b)
            wait_store(nmine - 2, rows_a, st_a)
            wait_store(nmine - 1, rows_b, st_b)

        plsc.subcore_barrier()
        pltpu.sync_copy(acc_sh.at[pl.ds(sid * rz, rz)],
                        acc_hbm.at[cid, pl.ds(sid * rz, rz)])
        if rtail:
            @pl.when(sid == 0)
            def _():
                pltpu.sync_copy(acc_sh.at[pl.ds(_NS * rz, rtail)],
                                acc_hbm.at[cid, pl.ds(_NS * rz, rtail)])

    return gs_k(g_lin, g_gat, s_idx, o_idx, b_vec, pred_ref)


def _prep_lo_pred(pred_vecs, w, n_lo, n_obj):
    d = pred_vecs.shape[1]
    ob = n_obj // _R

    def body(pred_ref, w_ref, g_ref):
        h = lax.dot_general(pred_ref[...], w_ref[...], (((1,), (1,)), ((), ())),
                            preferred_element_type=jnp.float32)
        g_ref[...] = h * 0.7071067811865476

    return pl.pallas_call(
        body,
        grid=(n_lo // _R - ob,),
        in_specs=[
            pl.BlockSpec((_R, d), lambda i: (i, 0)),
            pl.BlockSpec((d, d), lambda i: (0, 0)),
        ],
        out_specs=pl.BlockSpec((_R, d), lambda i: (i + ob, 0)),
        out_shape=jax.ShapeDtypeStruct((n_lo, d), jnp.float32),
    )(pred_vecs, w)


def _prep_obj(obj_vecs, w, cnt_t, prev):
    n_obj, d = obj_vecs.shape
    n_lo = prev.shape[0]

    def body(obj_ref, w_ref, cnt_ref, _prev_ref, g_ref):
        h = lax.dot_general(obj_ref[...], w_ref[...], (((1,), (1,)), ((), ())),
                            preferred_element_type=jnp.float32)
        cnt = jnp.sum(cnt_ref[...], axis=1, keepdims=True)
        g_ref[...] = h * lax.rsqrt(cnt + 2.0)

    return pl.pallas_call(
        body,
        grid=(n_obj // _R,),
        in_specs=[
            pl.BlockSpec((_R, d), lambda i: (i, 0)),
            pl.BlockSpec((d, d), lambda i: (0, 0)),
            pl.BlockSpec((_R, _NW), lambda i: (i, 0)),
            pl.BlockSpec((_R, d), lambda i: (i, 0)),
        ],
        out_specs=pl.BlockSpec((_R, d), lambda i: (i, 0)),
        out_shape=jax.ShapeDtypeStruct((n_lo, d), jnp.float32),
        input_output_aliases={3: 0},
    )(obj_vecs, w, cnt_t, prev)


def _prep_hi(pred_vecs, w, p0, n_hi, t_boundary):
    d = pred_vecs.shape[1]
    pb = p0 // _R
    tbb = t_boundary // _R

    def body(pred_ref, w_ref, g_ref):
        i = pl.program_id(0)
        h = lax.dot_general(pred_ref[...], w_ref[...], (((1,), (1,)), ((), ())),
                            preferred_element_type=jnp.float32)
        dinv = jnp.where(i + pb < tbb, jnp.float32(0.7071067811865476),
                         jnp.float32(1.0))
        g_ref[...] = h * dinv

    return pl.pallas_call(
        body,
        grid=(n_hi // _R,),
        in_specs=[
            pl.BlockSpec((_R, d), lambda i: (i + pb, 0)),
            pl.BlockSpec((d, d), lambda i: (0, 0)),
        ],
        out_specs=pl.BlockSpec((_R, d), lambda i: (i, 0)),
        out_shape=jax.ShapeDtypeStruct((n_hi, d), jnp.float32),
    )(pred_vecs, w)


def _combine_obj(g_lo, gath_head, accs, cnt_t, b2, n_obj):
    d = g_lo.shape[1]
    na = len(accs)

    def body(g_ref, gath_ref, *rest):
        acc_refs = rest[:na]
        cnt_ref, b_ref, out_ref = rest[na:]
        acc = acc_refs[0][0] + acc_refs[0][1]
        for a in acc_refs[1:]:
            acc = acc + a[0] + a[1]
        cnt = jnp.sum(cnt_ref[...], axis=1, keepdims=True)
        dinv = lax.rsqrt(cnt + 2.0)
        out_ref[...] = dinv * (g_ref[...] + gath_ref[...] + acc) + b_ref[...]

    return pl.pallas_call(
        body,
        grid=(n_obj // _R,),
        in_specs=[pl.BlockSpec((_R, d), lambda i: (i, 0)),
                  pl.BlockSpec((_R, d), lambda i: (i, 0))]
                 + [pl.BlockSpec((_NC, _R, d), lambda i: (0, i, 0))] * na
                 + [pl.BlockSpec((_R, _NW), lambda i: (i, 0)),
                    pl.BlockSpec((1, d), lambda i: (0, 0))],
        out_specs=pl.BlockSpec((_R, d), lambda i: (i, 0)),
        out_shape=jax.ShapeDtypeStruct((n_obj, d), jnp.float32),
    )(g_lo, gath_head, *accs, cnt_t, b2)


def _pred_tail(tail_g, b2, j0, nb, t):
    d = tail_g.shape[1]

    def body(g_ref, b_ref, out_ref):
        out_ref[...] = g_ref[...] + b_ref[...]

    return pl.pallas_call(
        body,
        grid=(nb,),
        in_specs=[pl.BlockSpec((_R, d), lambda i: (i, 0)),
                  pl.BlockSpec((1, d), lambda i: (0, 0))],
        out_specs=pl.BlockSpec((_R, d), lambda i: (i + j0, 0)),
        out_shape=jax.ShapeDtypeStruct((t, d), jnp.float32),
    )(tail_g, b2)


def _pred_fixup(g_lo, gath_head, b2, prev, n_obj, nraw):
    d = g_lo.shape[1]
    t = prev.shape[0]
    ob = n_obj // _R

    def body(g_ref, gh_ref, b_ref, prev_ref, out_ref):
        row = lax.broadcasted_iota(jnp.int32, (_R, 1), 0)
        fused = ((g_ref[...] + gh_ref[...]) * 0.7071067811865476
                 + b_ref[...])
        out_ref[...] = jnp.where(row < nraw, fused, prev_ref[...])

    return pl.pallas_call(
        body,
        grid=(1,),
        in_specs=[pl.BlockSpec((_R, d), lambda i: (ob, 0)),
                  pl.BlockSpec((_R, d), lambda i: (ob, 0)),
                  pl.BlockSpec((1, d), lambda i: (0, 0)),
                  pl.BlockSpec((_R, d), lambda i: (0, 0))],
        out_specs=pl.BlockSpec((_R, d), lambda i: (0, 0)),
        out_shape=jax.ShapeDtypeStruct((t, d), jnp.float32),
        input_output_aliases={3: 0},
    )(g_lo, gath_head, b2, prev)


def kernel(obj_vecs, pred_vecs, edges, W, b):
    n_obj, d = obj_vecs.shape
    t = pred_vecs.shape[0]
    sizes = [t // 4, t // 4, t // 4, t // 4]
    offs = [0]
    for sz in sizes:
        offs.append(offs[-1] + sz)
    cz = 64
    head = -(-n_obj // cz) * cz
    assert offs[-1] == t and sizes[0] > head
    assert all(sz % cz == 0 and sz % _R == 0 for sz in sizes)
    assert n_obj % _R == 0 and (t - n_obj) % _R == 0

    s_idx = edges[:, 0]
    o_idx = edges[:, 1]
    cnt32 = _hist_call(o_idx, n_obj)
    cnt_t = cnt32.T
    b2 = b.reshape(1, d)
    g0p = _prep_lo_pred(pred_vecs, W, sizes[0], n_obj)
    gps = [_prep_obj(obj_vecs, W, cnt_t, g0p)]
    for k in range(1, len(sizes)):
        gps.append(_prep_hi(pred_vecs, W, offs[k] - n_obj, sizes[k],
                            t - n_obj))
    tail_g = _prep_hi(pred_vecs, W, t - n_obj, n_obj, t - n_obj)
    tb = (t - n_obj) // _R
    ntail = n_obj // _R
    pred_init = _pred_tail(tail_g, b2, tb, ntail, t)
    pred_ref = jax.new_ref(pred_init)
    gath_head, acc0 = _gs_call(gps[0], gps[0], s_idx, o_idx, b, pred_ref,
                               n_obj, 0, 0, sizes[0], head)
    accs = [acc0]
    for k in range(1, len(sizes)):
        accs.append(_gs_call(gps[k], gps[0], s_idx, o_idx, b, pred_ref,
                             n_obj, offs[k], offs[k], sizes[k], 0))
    pred_val = pred_ref[...]
    out_pred = _pred_fixup(gps[0], gath_head, b2, pred_val, n_obj,
                           head - n_obj)
    out_obj = _combine_obj(gps[0], gath_head, accs, cnt_t, b2, n_obj)
    return out_obj, out_pred


# speedup vs baseline: 1.0333x; 1.0106x over previous
import functools

import jax
import jax.numpy as jnp
from jax import lax
from jax.experimental import pallas as pl
from jax.experimental.pallas import tpu as pltpu
from jax.experimental.pallas import tpu_sc as plsc

_NC, _NS, _L = 2, 16, 16
_NW = _NC * _NS
_R = 1000
_CB = 128


def _hist_call(o_idx, n_obj):
    t = o_idx.shape[0]
    nch = t // _CB
    full, extra = nch // _NW, nch % _NW
    mesh = plsc.VectorSubcoreMesh(core_axis_name="c", subcore_axis_name="s")

    @functools.partial(
        pl.kernel, mesh=mesh,
        out_type=jax.ShapeDtypeStruct((_NW, n_obj), jnp.float32),
        compiler_params=pltpu.CompilerParams(needs_layout_passes=False),
        scratch_types=[
            pltpu.VMEM((n_obj,), jnp.float32),
            pltpu.VMEM((_CB,), jnp.int32),
        ],
    )
    def hist_k(o_hbm, out_hbm, cnt_v, idx_v):
        wid = lax.axis_index("s") * _NC + lax.axis_index("c")
        zero16 = jnp.zeros((_L,), jnp.float32)

        def zb(i, c):
            cnt_v[pl.ds(i * _L, _L)] = zero16
            return c
        lax.fori_loop(0, n_obj // _L, zb, 0)

        one16 = jnp.ones((_L,), jnp.float32)
        nmine = full + jnp.where(wid < extra, 1, 0)

        def cb(j, c):
            eb = (wid + j * _NW) * _CB
            pltpu.sync_copy(o_hbm.at[pl.ds(eb, _CB)], idx_v)
            for i in range(_CB // _L):
                plsc.addupdate_scatter(cnt_v, [idx_v[pl.ds(i * _L, _L)]], one16)
            return c
        lax.fori_loop(0, nmine, cb, 0)
        pltpu.sync_copy(cnt_v, out_hbm.at[wid])

    return hist_k(o_idx)


def _gs_call(g_lin, g_gat, s_idx, o_idx, b_vec, pred_ref,
             n_obj, ebase, lin_base, half_t, head):
    d = g_lin.shape[1]
    t = half_t
    cz = 64
    nch = t // cz
    full, extra = nch // _NW, nch % _NW
    rz = 624
    rtail = n_obj - _NS * rz
    mesh = plsc.VectorSubcoreMesh(core_axis_name="c", subcore_axis_name="s")
    if head:
        out_type = (jax.ShapeDtypeStruct((n_obj + _R, d), jnp.float32),
                    jax.ShapeDtypeStruct((_NC, n_obj, d), jnp.float32))
    else:
        out_type = jax.ShapeDtypeStruct((_NC, n_obj, d), jnp.float32)

    @functools.partial(
        pl.kernel, mesh=mesh,
        out_type=out_type,
        compiler_params=pltpu.CompilerParams(needs_layout_passes=False),
        scratch_types=[
            pltpu.VMEM((cz,), jnp.int32),
            pltpu.VMEM((cz,), jnp.int32),
            pltpu.VMEM((cz, d), jnp.float32),
            pltpu.VMEM((cz, d), jnp.float32),
            pltpu.VMEM((cz,), jnp.int32),
            pltpu.VMEM((cz,), jnp.int32),
            pltpu.VMEM((cz, d), jnp.float32),
            pltpu.VMEM((cz, d), jnp.float32),
            pltpu.VMEM((d,), jnp.float32),
            pltpu.VMEM_SHARED((n_obj, d), jnp.float32),
            pltpu.SemaphoreType.DMA,
            pltpu.SemaphoreType.DMA,
            pltpu.SemaphoreType.DMA,
            pltpu.SemaphoreType.DMA,
            pltpu.SemaphoreType.DMA,
            pltpu.SemaphoreType.DMA,
        ],
    )
    def gs_k(glin_hbm, ggat_hbm, s_hbm, o_hbm, b_hbm, pred_hbm, *rest):
        if head:
            (gh_hbm, acc_hbm,
             sidx_a, oidx_a, grows_a, rows_a,
             sidx_b, oidx_b, grows_b, rows_b,
             b_v, acc_sh, ld_a, ld_b, gat, st_a, st_b, sc) = rest
        else:
            (acc_hbm,
             sidx_a, oidx_a, grows_a, rows_a,
             sidx_b, oidx_b, grows_b, rows_b,
             b_v, acc_sh, ld_a, ld_b, gat, st_a, st_b, sc) = rest
            gh_hbm = None
        cid = lax.axis_index("c")
        sid = lax.axis_index("s")
        wid = sid * _NC + cid

        pltpu.sync_copy(b_hbm, b_v)
        b8 = [b_v[pl.ds(i * _L, _L)] for i in range(d // _L)]

        zf = jnp.zeros((_L,), jnp.float32)

        def zb(r, c):
            for i in range(d // _L):
                grows_a[r, pl.ds(i * _L, _L)] = zf
            return c
        lax.fori_loop(0, cz, zb, 0)
        nzc, ztail = rz // cz, rz % cz
        zhs = [pltpu.async_copy(grows_a,
                                acc_sh.at[pl.ds(sid * rz + z * cz, cz)], ld_b)
               for z in range(nzc)]
        if ztail:
            zhs.append(pltpu.async_copy(
                grows_a.at[pl.ds(0, ztail)],
                acc_sh.at[pl.ds(sid * rz + nzc * cz, ztail)], ld_b))
        if rtail:
            @pl.when(sid == 0)
            def _():
                pltpu.async_copy(grows_a.at[pl.ds(0, rtail)],
                                 acc_sh.at[pl.ds(_NS * rz, rtail)],
                                 ld_b).wait()
        for h in zhs:
            h.wait()
        plsc.subcore_barrier()

        nmine = full + jnp.where(wid < extra, 1, 0)

        def loads(j, sidx_v, oidx_v, grows_v, ld):
            eb = (wid + j * _NW) * cz
            pltpu.async_copy(s_hbm.at[pl.ds(ebase + eb, cz)], sidx_v, ld)
            pltpu.async_copy(o_hbm.at[pl.ds(ebase + eb, cz)], oidx_v, ld)
            pltpu.async_copy(glin_hbm.at[pl.ds(ebase - lin_base + eb, cz)],
                             grows_v, ld)

        def wait_loads(j, sidx_v, oidx_v, grows_v, ld):
            eb = (wid + j * _NW) * cz
            pltpu.make_async_copy(
                s_hbm.at[pl.ds(ebase + eb, cz)], sidx_v, ld).wait()
            pltpu.make_async_copy(
                o_hbm.at[pl.ds(ebase + eb, cz)], oidx_v, ld).wait()
            pltpu.make_async_copy(
                glin_hbm.at[pl.ds(ebase - lin_base + eb, cz)],
                grows_v, ld).wait()

        def fuse(rows_v, grows_v):
            def fb(r, c):
                for i in range(d // _L):
                    sl = pl.ds(i * _L, _L)
                    rows_v[r, sl] = ((rows_v[r, sl] + grows_v[r, sl])
                                     * 0.7071067811865476 + b8[i])
                return c
            lax.fori_loop(0, cz, fb, 0)

        def finish(j, sidx_v, oidx_v, grows_v, rows_v, st):
            eb = (wid + j * _NW) * cz
            hs = pltpu.async_copy(grows_v, acc_sh.at[oidx_v], sc, add=True)
            if head:
                @pl.when(eb < head)
                def _():
                    pltpu.async_copy(rows_v, gh_hbm.at[pl.ds(eb, cz)], st)

                @pl.when(eb >= head)
                def _():
                    fuse(rows_v, grows_v)
                    pltpu.async_copy(
                        rows_v,
                        pred_hbm.at[pl.ds(ebase + eb - n_obj, cz)], st)
            else:
                fuse(rows_v, grows_v)
                pltpu.async_copy(
                    rows_v, pred_hbm.at[pl.ds(ebase + eb - n_obj, cz)], st)
            hs.wait()

        def wait_store(j, rows_v, st):
            eb = (wid + j * _NW) * cz
            if head:
                @pl.when(eb < head)
                def _():
                    pltpu.make_async_copy(
                        rows_v, gh_hbm.at[pl.ds(eb, cz)], st).wait()

                @pl.when(eb >= head)
                def _():
                    pltpu.make_async_copy(
                        rows_v,
                        pred_hbm.at[pl.ds(ebase + eb - n_obj, cz)], st).wait()
            else:
                pltpu.make_async_copy(
                    rows_v,
                    pred_hbm.at[pl.ds(ebase + eb - n_obj, cz)], st).wait()

        def stage(j, cur, oth):
            (sidx_c, oidx_c, grows_c, rows_c, ld_c, st_c) = cur
            (sidx_o, oidx_o, grows_o, rows_o, ld_o, st_o) = oth

            @pl.when(j >= 2)
            def _():
                wait_store(j - 2, rows_c, st_c)

            wait_loads(j, sidx_c, oidx_c, grows_c, ld_c)
            hg = pltpu.async_copy(ggat_hbm.at[sidx_c], rows_c, gat)

            @pl.when(j >= 1)
            def _():
                finish(j - 1, sidx_o, oidx_o, grows_o, rows_o, st_o)

            @pl.when(j + 1 < nmine)
            def _():
                loads(j + 1, sidx_o, oidx_o, grows_o, ld_o)

            hg.wait()

        buf_a = (sidx_a, oidx_a, grows_a, rows_a, ld_a, st_a)
        buf_b = (sidx_b, oidx_b, grows_b, rows_b, ld_b, st_b)

        loads(0, sidx_a, oidx_a, grows_a, ld_a)

        def cb(j, c):
            @pl.when((j & 1) == 0)
            def _():
                stage(j, buf_a, buf_b)

            @pl.when((j & 1) == 1)
            def _():
                stage(j, buf_b, buf_a)
            return c
        lax.fori_loop(0, nmine, cb, 0)

        @pl.when(((nmine - 1) & 1) == 0)
        def _():
            finish(nmine - 1, sidx_a, oidx_a, grows_a, rows_a, st_a)
            wait_store(nmine - 2, rows_b, st_b)
            wait_store(nmine - 1, rows_a, st_a)

        @pl.when(((nmine - 1) & 1) == 1)
        def _():
            finish(nmine - 1, sidx_b, oidx_b, grows_b, rows_b, st_b)
            wait_store(nmine - 2, rows_a, st_a)
            wait_store(nmine - 1, rows_b, st_b)

        plsc.subcore_barrier()
        pltpu.sync_copy(acc_sh.at[pl.ds(sid * rz, rz)],
                        acc_hbm.at[cid, pl.ds(sid * rz, rz)])
        if rtail:
            @pl.when(sid == 0)
            def _():
                pltpu.sync_copy(acc_sh.at[pl.ds(_NS * rz, rtail)],
                                acc_hbm.at[cid, pl.ds(_NS * rz, rtail)])

    return gs_k(g_lin, g_gat, s_idx, o_idx, b_vec, pred_ref)


def _prep_lo_pred(pred_vecs, w, n_lo, n_obj):
    d = pred_vecs.shape[1]
    ob = n_obj // _R

    def body(pred_ref, w_ref, g_ref):
        h = lax.dot_general(pred_ref[...], w_ref[...], (((1,), (1,)), ((), ())),
                            preferred_element_type=jnp.float32)
        g_ref[...] = h * 0.7071067811865476

    return pl.pallas_call(
        body,
        grid=(n_lo // _R - ob,),
        in_specs=[
            pl.BlockSpec((_R, d), lambda i: (i, 0)),
            pl.BlockSpec((d, d), lambda i: (0, 0)),
        ],
        out_specs=pl.BlockSpec((_R, d), lambda i: (i + ob, 0)),
        out_shape=jax.ShapeDtypeStruct((n_lo, d), jnp.float32),
    )(pred_vecs, w)


def _prep_obj(obj_vecs, w, cnt_t, prev):
    n_obj, d = obj_vecs.shape
    n_lo = prev.shape[0]

    def body(obj_ref, w_ref, cnt_ref, _prev_ref, g_ref):
        h = lax.dot_general(obj_ref[...], w_ref[...], (((1,), (1,)), ((), ())),
                            preferred_element_type=jnp.float32)
        cnt = jnp.sum(cnt_ref[...], axis=1, keepdims=True)
        g_ref[...] = h * lax.rsqrt(cnt + 2.0)

    return pl.pallas_call(
        body,
        grid=(n_obj // _R,),
        in_specs=[
            pl.BlockSpec((_R, d), lambda i: (i, 0)),
            pl.BlockSpec((d, d), lambda i: (0, 0)),
            pl.BlockSpec((_R, _NW), lambda i: (i, 0)),
            pl.BlockSpec((_R, d), lambda i: (i, 0)),
        ],
        out_specs=pl.BlockSpec((_R, d), lambda i: (i, 0)),
        out_shape=jax.ShapeDtypeStruct((n_lo, d), jnp.float32),
        input_output_aliases={3: 0},
    )(obj_vecs, w, cnt_t, prev)


def _prep_hi(pred_vecs, w, p0, n_hi, t_boundary):
    d = pred_vecs.shape[1]
    pb = p0 // _R
    tbb = t_boundary // _R

    def body(pred_ref, w_ref, g_ref):
        i = pl.program_id(0)
        h = lax.dot_general(pred_ref[...], w_ref[...], (((1,), (1,)), ((), ())),
                            preferred_element_type=jnp.float32)
        dinv = jnp.where(i + pb < tbb, jnp.float32(0.7071067811865476),
                         jnp.float32(1.0))
        g_ref[...] = h * dinv

    return pl.pallas_call(
        body,
        grid=(n_hi // _R,),
        in_specs=[
            pl.BlockSpec((_R, d), lambda i: (i + pb, 0)),
            pl.BlockSpec((d, d), lambda i: (0, 0)),
        ],
        out_specs=pl.BlockSpec((_R, d), lambda i: (i, 0)),
        out_shape=jax.ShapeDtypeStruct((n_hi, d), jnp.float32),
    )(pred_vecs, w)


def _combine_obj(g_lo, gath_head, accs, cnt_t, b2, n_obj):
    d = g_lo.shape[1]
    na = len(accs)

    def body(g_ref, gath_ref, *rest):
        acc_refs = rest[:na]
        cnt_ref, b_ref, out_ref = rest[na:]
        acc = acc_refs[0][0] + acc_refs[0][1]
        for a in acc_refs[1:]:
            acc = acc + a[0] + a[1]
        cnt = jnp.sum(cnt_ref[...], axis=1, keepdims=True)
        dinv = lax.rsqrt(cnt + 2.0)
        out_ref[...] = dinv * (g_ref[...] + gath_ref[...] + acc) + b_ref[...]

    return pl.pallas_call(
        body,
        grid=(n_obj // _R,),
        in_specs=[pl.BlockSpec((_R, d), lambda i: (i, 0)),
                  pl.BlockSpec((_R, d), lambda i: (i, 0))]
                 + [pl.BlockSpec((_NC, _R, d), lambda i: (0, i, 0))] * na
                 + [pl.BlockSpec((_R, _NW), lambda i: (i, 0)),
                    pl.BlockSpec((1, d), lambda i: (0, 0))],
        out_specs=pl.BlockSpec((_R, d), lambda i: (i, 0)),
        out_shape=jax.ShapeDtypeStruct((n_obj, d), jnp.float32),
    )(g_lo, gath_head, *accs, cnt_t, b2)


def _pred_tail(g_hi, b2, prev, gb0, j0, nb):
    d = g_hi.shape[1]
    t = prev.shape[0]

    def body(g_ref, b_ref, _prev_ref, out_ref):
        out_ref[...] = g_ref[...] + b_ref[...]

    return pl.pallas_call(
        body,
        grid=(nb,),
        in_specs=[pl.BlockSpec((_R, d), lambda i: (i + gb0, 0)),
                  pl.BlockSpec((1, d), lambda i: (0, 0)),
                  pl.BlockSpec((_R, d), lambda i: (i + j0, 0))],
        out_specs=pl.BlockSpec((_R, d), lambda i: (i + j0, 0)),
        out_shape=jax.ShapeDtypeStruct((t, d), jnp.float32),
        input_output_aliases={2: 0},
    )(g_hi, b2, prev)


def _pred_fixup(g_lo, gath_head, b2, prev, n_obj, nraw):
    d = g_lo.shape[1]
    t = prev.shape[0]
    ob = n_obj // _R

    def body(g_ref, gh_ref, b_ref, prev_ref, out_ref):
        row = lax.broadcasted_iota(jnp.int32, (_R, 1), 0)
        fused = ((g_ref[...] + gh_ref[...]) * 0.7071067811865476
                 + b_ref[...])
        out_ref[...] = jnp.where(row < nraw, fused, prev_ref[...])

    return pl.pallas_call(
        body,
        grid=(1,),
        in_specs=[pl.BlockSpec((_R, d), lambda i: (ob, 0)),
                  pl.BlockSpec((_R, d), lambda i: (ob, 0)),
                  pl.BlockSpec((1, d), lambda i: (0, 0)),
                  pl.BlockSpec((_R, d), lambda i: (0, 0))],
        out_specs=pl.BlockSpec((_R, d), lambda i: (0, 0)),
        out_shape=jax.ShapeDtypeStruct((t, d), jnp.float32),
        input_output_aliases={3: 0},
    )(g_lo, gath_head, b2, prev)


def kernel(obj_vecs, pred_vecs, edges, W, b):
    n_obj, d = obj_vecs.shape
    t = pred_vecs.shape[0]
    nsplit = 4
    q = t // nsplit
    cz = 64
    head = -(-n_obj // cz) * cz
    assert q % cz == 0 and q > head and t % nsplit == 0
    assert n_obj % _R == 0 and q % _R == 0 and (t - n_obj) % _R == 0

    s_idx = edges[:, 0]
    o_idx = edges[:, 1]
    cnt32 = _hist_call(o_idx, n_obj)
    cnt_t = cnt32.T
    g0p = _prep_lo_pred(pred_vecs, W, q, n_obj)
    gps = [_prep_obj(obj_vecs, W, cnt_t, g0p)]
    for k in range(1, nsplit):
        nk = q if k < nsplit - 1 else q + n_obj
        gps.append(_prep_hi(pred_vecs, W, k * q - n_obj, nk, t - n_obj))
    pred_ref = jax.empty_ref(jax.ShapeDtypeStruct((t, d), jnp.float32))
    gath_head, acc0 = _gs_call(gps[0], gps[0], s_idx, o_idx, b, pred_ref,
                               n_obj, 0, 0, q, head)
    accs = [acc0]
    for k in range(1, nsplit):
        accs.append(_gs_call(gps[k], gps[0], s_idx, o_idx, b, pred_ref,
                             n_obj, k * q, k * q, q, 0))
    pred_val = pred_ref[...]
    b2 = b.reshape(1, d)
    tb = (t - n_obj) // _R
    ntail = n_obj // _R
    gb0 = (t - (nsplit - 1) * q) // _R
    p0 = _pred_tail(gps[-1], b2, pred_val, gb0, tb, ntail)
    out_pred = _pred_fixup(gps[0], gath_head, b2, p0, n_obj, head - n_obj)
    out_obj = _combine_obj(gps[0], gath_head, accs, cnt_t, b2, n_obj)
    return out_obj, out_pred
